# baseline (speedup 1.0000x reference)
import functools

import jax
import jax.numpy as jnp
from jax import lax
from jax.experimental import pallas as pl
from jax.experimental.pallas import tpu as pltpu
from jax.experimental.pallas import tpu_sc as plsc

NUM_TASKS = 100000
NUM_EXPERTS = 128
BATCH = 16384

_NC = 2
_NS = 16
_NW = _NC * _NS
_B_PER_W = BATCH // _NW
_CHUNK = 128
_NCHUNK = _B_PER_W // _CHUNK
_L = 16


def _transpose_chunk(rows_c, trans_c, iotas):
    def e_body(e, carry):
        col = jnp.full((_L,), e, jnp.int32)
        for g in range(_CHUNK // _L):
            v = plsc.load_gather(rows_c, [iotas[g], col])
            trans_c[e, pl.ds(g * _L, _L)] = v
        return carry

    lax.fori_loop(0, NUM_EXPERTS, e_body, 0)


def _sc_gather_transpose(table, idx3):
    mesh = plsc.VectorSubcoreMesh(core_axis_name="c", subcore_axis_name="s")

    @functools.partial(
        pl.kernel,
        out_type=jax.ShapeDtypeStruct(
            (NUM_EXPERTS, BATCH // _CHUNK, _CHUNK), jnp.float32
        ),
        mesh=mesh,
        compiler_params=pltpu.CompilerParams(needs_layout_passes=False),
        scratch_types=[
            pltpu.VMEM((_NCHUNK, _CHUNK), jnp.int32),
            pltpu.VMEM((_CHUNK, NUM_EXPERTS), jnp.float32),
            pltpu.VMEM((_CHUNK, NUM_EXPERTS), jnp.float32),
            pltpu.VMEM((_CHUNK, NUM_EXPERTS), jnp.float32),
            pltpu.VMEM((_CHUNK, NUM_EXPERTS), jnp.float32),
            pltpu.VMEM((NUM_EXPERTS, _CHUNK), jnp.float32),
            pltpu.VMEM((NUM_EXPERTS, _CHUNK), jnp.float32),
            pltpu.SemaphoreType.DMA,
            pltpu.SemaphoreType.DMA,
            pltpu.SemaphoreType.DMA,
            pltpu.SemaphoreType.DMA,
            pltpu.SemaphoreType.DMA,
            pltpu.SemaphoreType.DMA,
        ],
    )
    def k(table_hbm, idx_hbm, out_hbm, idx_v, r0, r1, r2, r3, t0, t1,
          g0, g1, g2, g3, w0, w1):
        rows = [r0, r1, r2, r3]
        trans = [t0, t1]
        gsems = [g0, g1, g2, g3]
        wsems = [w0, w1]
        wid = lax.axis_index("s") * _NC + lax.axis_index("c")
        pltpu.sync_copy(idx_hbm.at[wid], idx_v)
        gathers = [
            pltpu.async_copy(table_hbm.at[idx_v.at[c]], rows[c], gsems[c])
            for c in range(_NCHUNK)
        ]
        iotas = [lax.iota(jnp.int32, _L) + g * _L for g in range(_CHUNK // _L)]
        writes = [None, None]
        for c in range(_NCHUNK):
            gathers[c].wait()
            buf = c % 2
            if writes[buf] is not None:
                writes[buf].wait()
            _transpose_chunk(rows[c], trans[buf], iotas)
            writes[buf] = pltpu.async_copy(
                trans[buf],
                out_hbm.at[:, wid * _NCHUNK + c, :],
                wsems[buf],
            )
        writes[0].wait()
        writes[1].wait()

    return k(table, idx3)


def kernel(env_index, task_index_to_mask):
    idx = env_index.reshape(_NW, _NCHUNK, _CHUNK).astype(jnp.int32)
    out = _sc_gather_transpose(task_index_to_mask, idx)
    return out.reshape(NUM_EXPERTS, BATCH)[:, :, None]


# speedup vs baseline: 1.4503x; 1.4503x over previous
import functools

import jax
import jax.numpy as jnp
from jax import lax
from jax.experimental import pallas as pl
from jax.experimental.pallas import tpu as pltpu
from jax.experimental.pallas import tpu_sc as plsc

NUM_TASKS = 100000
NUM_EXPERTS = 128
BATCH = 16384

_NC = 2
_NS = 16
_NW = _NC * _NS
_B_PER_W = BATCH // _NW
_CHUNK = 128
_NCHUNK = _B_PER_W // _CHUNK
_L = 16


def _transpose_chunk(rows_c, trans_c, iotas):
    @plsc.parallel_loop(0, NUM_EXPERTS, unroll=4)
    def e_body(e):
        col = jnp.full((_L,), e, jnp.int32)
        for g in range(_CHUNK // _L):
            v = plsc.load_gather(rows_c, [iotas[g], col])
            trans_c[e, pl.ds(g * _L, _L)] = v


def _sc_gather_transpose(table, idx3):
    mesh = plsc.VectorSubcoreMesh(core_axis_name="c", subcore_axis_name="s")

    @functools.partial(
        pl.kernel,
        out_type=jax.ShapeDtypeStruct(
            (NUM_EXPERTS, BATCH // _CHUNK, _CHUNK), jnp.float32
        ),
        mesh=mesh,
        compiler_params=pltpu.CompilerParams(needs_layout_passes=False),
        scratch_types=[
            pltpu.VMEM((_NCHUNK, _CHUNK), jnp.int32),
            pltpu.VMEM((_CHUNK, NUM_EXPERTS), jnp.float32),
            pltpu.VMEM((_CHUNK, NUM_EXPERTS), jnp.float32),
            pltpu.VMEM((_CHUNK, NUM_EXPERTS), jnp.float32),
            pltpu.VMEM((_CHUNK, NUM_EXPERTS), jnp.float32),
            pltpu.VMEM((NUM_EXPERTS, _CHUNK), jnp.float32),
            pltpu.VMEM((NUM_EXPERTS, _CHUNK), jnp.float32),
            pltpu.SemaphoreType.DMA,
            pltpu.SemaphoreType.DMA,
            pltpu.SemaphoreType.DMA,
            pltpu.SemaphoreType.DMA,
            pltpu.SemaphoreType.DMA,
            pltpu.SemaphoreType.DMA,
        ],
    )
    def k(table_hbm, idx_hbm, out_hbm, idx_v, r0, r1, r2, r3, t0, t1,
          g0, g1, g2, g3, w0, w1):
        rows = [r0, r1, r2, r3]
        trans = [t0, t1]
        gsems = [g0, g1, g2, g3]
        wsems = [w0, w1]
        wid = lax.axis_index("s") * _NC + lax.axis_index("c")
        pltpu.sync_copy(idx_hbm.at[wid], idx_v)
        gathers = [
            pltpu.async_copy(table_hbm.at[idx_v.at[c]], rows[c], gsems[c])
            for c in range(_NCHUNK)
        ]
        iotas = [lax.iota(jnp.int32, _L) + g * _L for g in range(_CHUNK // _L)]
        writes = [None, None]
        for c in range(_NCHUNK):
            gathers[c].wait()
            buf = c % 2
            if writes[buf] is not None:
                writes[buf].wait()
            _transpose_chunk(rows[c], trans[buf], iotas)
            writes[buf] = pltpu.async_copy(
                trans[buf],
                out_hbm.at[:, wid * _NCHUNK + c, :],
                wsems[buf],
            )
        writes[0].wait()
        writes[1].wait()

    return k(table, idx3)


def kernel(env_index, task_index_to_mask):
    idx = env_index.reshape(_NW, _NCHUNK, _CHUNK).astype(jnp.int32)
    out = _sc_gather_transpose(task_index_to_mask, idx)
    return out.reshape(NUM_EXPERTS, BATCH)[:, :, None]
